# Initial kernel scaffold; baseline (speedup 1.0000x reference)
import functools

import jax
import jax.numpy as jnp
from jax import lax
from jax.experimental import pallas as pl
from jax.experimental.pallas import tpu as pltpu
from jax.experimental.pallas import tpu_sc as plsc

_E = 327680
_D = 64
_LANES = 16
_NW = 32
_EPW = _E // _NW
_C = 512
_NCHUNK = _EPW // _C
_IDX_ROWS = _EPW // 128


def _sc_body(src_hbm, dst_hbm, u_hbm, v_hbm, out_hbm,
             sidx, didx, ubuf, vbuf, obuf, sem):
    nc = 2
    wid = lax.axis_index("s") * nc + lax.axis_index("c")
    row0 = wid * _IDX_ROWS

    pltpu.sync_copy(src_hbm.at[pl.ds(row0, _IDX_ROWS)], sidx)
    pltpu.sync_copy(dst_hbm.at[pl.ds(row0, _IDX_ROWS)], didx)

    iota = lax.iota(jnp.int32, _LANES)

    def chunk_body(g, _):
        j0 = g * (_C // 128)
        for j in range(_C // 128):
            pltpu.make_async_copy(
                u_hbm.at[sidx.at[j0 + j]],
                ubuf.at[pl.ds(j * 128, 128)], sem).start()
            pltpu.make_async_copy(
                v_hbm.at[didx.at[j0 + j]],
                vbuf.at[pl.ds(j * 128, 128)], sem).start()
        for j in range(_C // 128):
            pltpu.make_async_copy(
                u_hbm.at[sidx.at[j0 + j]],
                ubuf.at[pl.ds(j * 128, 128)], sem).wait()
            pltpu.make_async_copy(
                v_hbm.at[didx.at[j0 + j]],
                vbuf.at[pl.ds(j * 128, 128)], sem).wait()

        out_base = g * _C

        def grp_body(grp, _):
            rows = grp * _LANES + iota

            def d_body(t, acc):
                for k in range(4):
                    d = t * 4 + k
                    cols = jnp.full((_LANES,), d, jnp.int32)
                    u = plsc.load_gather(ubuf, [rows, cols])
                    v = plsc.load_gather(vbuf, [rows, cols])
                    x = u * v
                    acc = acc + 1.0 / (1.0 + jnp.exp(-x))
                return acc

            acc = lax.fori_loop(0, _D // 4, d_body,
                                jnp.zeros((_LANES,), jnp.float32))
            obuf[pl.ds(out_base + grp * _LANES, _LANES)] = acc
            return 0

        lax.fori_loop(0, _C // _LANES, grp_body, 0)
        return 0

    lax.fori_loop(0, _NCHUNK, chunk_body, 0)

    pltpu.sync_copy(obuf, out_hbm.at[pl.ds(wid * _EPW, _EPW)])


@jax.jit
def _sc_call(src2d, dst2d, u, v):
    f = pl.kernel(
        _sc_body,
        out_type=jax.ShapeDtypeStruct((_E,), jnp.float32),
        mesh=plsc.VectorSubcoreMesh(core_axis_name="c", subcore_axis_name="s"),
        scratch_types=[
            pltpu.VMEM((_IDX_ROWS, 128), jnp.int32),
            pltpu.VMEM((_IDX_ROWS, 128), jnp.int32),
            pltpu.VMEM((_C, _D), jnp.float32),
            pltpu.VMEM((_C, _D), jnp.float32),
            pltpu.VMEM((_EPW,), jnp.float32),
            pltpu.SemaphoreType.DMA,
        ],
    )
    return f(src2d, dst2d, u, v)


def kernel(edge_index, U, V):
    src = edge_index[0].astype(jnp.int32).reshape(_E // 128, 128)
    dst = edge_index[1].astype(jnp.int32).reshape(_E // 128, 128)
    return _sc_call(src, dst, U, V)


# speedup vs baseline: 2.8137x; 2.8137x over previous
import functools

import jax
import jax.numpy as jnp
from jax import lax
from jax.experimental import pallas as pl
from jax.experimental.pallas import tpu as pltpu
from jax.experimental.pallas import tpu_sc as plsc

_E = 327680
_D = 64
_LANES = 16
_NW = 32
_EPW = _E // _NW
_C = 512
_NCHUNK = _EPW // _C
_IDX_ROWS = _EPW // 128


def _sc_body(src_hbm, dst_hbm, u_hbm, v_hbm, out_hbm,
             sidx, didx, ubuf, vbuf, obuf, sem):
    nc = 2
    wid = lax.axis_index("s") * nc + lax.axis_index("c")
    row0 = wid * _IDX_ROWS

    pltpu.sync_copy(src_hbm.at[pl.ds(row0, _IDX_ROWS)], sidx)
    pltpu.sync_copy(dst_hbm.at[pl.ds(row0, _IDX_ROWS)], didx)

    iota = lax.iota(jnp.int32, _LANES)

    def chunk_body(g, _):
        j0 = g * (_C // 128)
        for j in range(_C // 128):
            pltpu.make_async_copy(
                u_hbm.at[sidx.at[j0 + j]],
                ubuf.at[pl.ds(j * 128, 128)], sem).start()
            pltpu.make_async_copy(
                v_hbm.at[didx.at[j0 + j]],
                vbuf.at[pl.ds(j * 128, 128)], sem).start()
        for j in range(_C // 128):
            pltpu.make_async_copy(
                u_hbm.at[sidx.at[j0 + j]],
                ubuf.at[pl.ds(j * 128, 128)], sem).wait()
            pltpu.make_async_copy(
                v_hbm.at[didx.at[j0 + j]],
                vbuf.at[pl.ds(j * 128, 128)], sem).wait()

        out_base = g * _C

        def grp_body(grp, _):
            rows = grp * _LANES + iota

            def d_body(t, acc):
                for k in range(4):
                    d = t * 4 + k
                    cols = jnp.full((_LANES,), d, jnp.int32)
                    u = plsc.load_gather(ubuf, [rows, cols])
                    v = plsc.load_gather(vbuf, [rows, cols])
                    x = u * v
                    acc = acc + 1.0 / (1.0 + jnp.exp(-x))
                return acc

            acc = lax.fori_loop(0, _D // 4, d_body,
                                jnp.zeros((_LANES,), jnp.float32))
            obuf[pl.ds(out_base + grp * _LANES, _LANES)] = acc
            return 0

        lax.fori_loop(0, _C // _LANES, grp_body, 0)
        return 0

    lax.fori_loop(0, _NCHUNK, chunk_body, 0)

    pltpu.sync_copy(obuf, out_hbm.at[pl.ds(wid * _EPW, _EPW)])


@jax.jit
def _sc_call(src2d, dst2d, u, v):
    f = pl.kernel(
        _sc_body,
        out_type=jax.ShapeDtypeStruct((_E,), jnp.float32),
        mesh=plsc.VectorSubcoreMesh(core_axis_name="c", subcore_axis_name="s"),
        compiler_params=pltpu.CompilerParams(
            needs_layout_passes=False, use_tc_tiling_on_sc=False),
        scratch_types=[
            pltpu.VMEM((_IDX_ROWS, 128), jnp.int32),
            pltpu.VMEM((_IDX_ROWS, 128), jnp.int32),
            pltpu.VMEM((_C, _D), jnp.float32),
            pltpu.VMEM((_C, _D), jnp.float32),
            pltpu.VMEM((_EPW,), jnp.float32),
            pltpu.SemaphoreType.DMA,
        ],
    )
    return f(src2d, dst2d, u, v)


def kernel(edge_index, U, V):
    src = edge_index[0].astype(jnp.int32).reshape(_E // 128, 128)
    dst = edge_index[1].astype(jnp.int32).reshape(_E // 128, 128)
    return _sc_call(src, dst, U, V)


# speedup vs baseline: 3.0197x; 1.0732x over previous
import functools

import jax
import jax.numpy as jnp
from jax import lax
from jax.experimental import pallas as pl
from jax.experimental.pallas import tpu as pltpu
from jax.experimental.pallas import tpu_sc as plsc

_E = 327680
_D = 64
_LANES = 16
_NW = 32
_EPW = _E // _NW
_C = 256
_JPC = _C // 128
_NCHUNK = _EPW // _C
_IDX_ROWS = _EPW // 128


def _sc_body(src_hbm, dst_hbm, u_hbm, v_hbm, out_hbm,
             sidx, didx, ubuf0, vbuf0, ubuf1, vbuf1, obuf, sem0, sem1):
    nc = 2
    wid = lax.axis_index("s") * nc + lax.axis_index("c")
    row0 = wid * _IDX_ROWS

    pltpu.sync_copy(src_hbm.at[pl.ds(row0, _IDX_ROWS)], sidx)
    pltpu.sync_copy(dst_hbm.at[pl.ds(row0, _IDX_ROWS)], didx)

    iota = lax.iota(jnp.int32, _LANES)

    def fire(g, ubuf, vbuf, sem):
        j0 = g * _JPC
        for j in range(_JPC):
            pltpu.make_async_copy(
                u_hbm.at[sidx.at[j0 + j]],
                ubuf.at[pl.ds(j * 128, 128)], sem).start()
            pltpu.make_async_copy(
                v_hbm.at[didx.at[j0 + j]],
                vbuf.at[pl.ds(j * 128, 128)], sem).start()

    def wait(g, ubuf, vbuf, sem):
        j0 = g * _JPC
        for j in range(_JPC):
            pltpu.make_async_copy(
                u_hbm.at[sidx.at[j0 + j]],
                ubuf.at[pl.ds(j * 128, 128)], sem).wait()
            pltpu.make_async_copy(
                v_hbm.at[didx.at[j0 + j]],
                vbuf.at[pl.ds(j * 128, 128)], sem).wait()

    def compute(g, ubuf, vbuf):
        out_base = g * _C

        def grp_body(grp, _):
            rows = grp * _LANES + iota

            def d_body(t, acc):
                for k in range(4):
                    d = t * 4 + k
                    cols = jnp.full((_LANES,), d, jnp.int32)
                    u = plsc.load_gather(ubuf, [rows, cols])
                    v = plsc.load_gather(vbuf, [rows, cols])
                    x = u * v
                    acc = acc + 1.0 / (1.0 + jnp.exp(-x))
                return acc

            acc = lax.fori_loop(0, _D // 4, d_body,
                                jnp.zeros((_LANES,), jnp.float32))
            obuf[pl.ds(out_base + grp * _LANES, _LANES)] = acc
            return 0

        lax.fori_loop(0, _C // _LANES, grp_body, 0)

    fire(0, ubuf0, vbuf0, sem0)

    def pair_body(gp, _):
        g0 = gp * 2
        fire(g0 + 1, ubuf1, vbuf1, sem1)
        wait(g0, ubuf0, vbuf0, sem0)
        compute(g0, ubuf0, vbuf0)

        @pl.when(g0 + 2 < _NCHUNK)
        def _():
            fire(g0 + 2, ubuf0, vbuf0, sem0)

        wait(g0 + 1, ubuf1, vbuf1, sem1)
        compute(g0 + 1, ubuf1, vbuf1)
        return 0

    lax.fori_loop(0, _NCHUNK // 2, pair_body, 0)

    pltpu.sync_copy(obuf, out_hbm.at[pl.ds(wid * _EPW, _EPW)])


@jax.jit
def _sc_call(src2d, dst2d, u, v):
    f = pl.kernel(
        _sc_body,
        out_type=jax.ShapeDtypeStruct((_E,), jnp.float32),
        mesh=plsc.VectorSubcoreMesh(core_axis_name="c", subcore_axis_name="s"),
        compiler_params=pltpu.CompilerParams(
            needs_layout_passes=False, use_tc_tiling_on_sc=False),
        scratch_types=[
            pltpu.VMEM((_IDX_ROWS, 128), jnp.int32),
            pltpu.VMEM((_IDX_ROWS, 128), jnp.int32),
            pltpu.VMEM((_C, _D), jnp.float32),
            pltpu.VMEM((_C, _D), jnp.float32),
            pltpu.VMEM((_C, _D), jnp.float32),
            pltpu.VMEM((_C, _D), jnp.float32),
            pltpu.VMEM((_EPW,), jnp.float32),
            pltpu.SemaphoreType.DMA,
            pltpu.SemaphoreType.DMA,
        ],
    )
    return f(src2d, dst2d, u, v)


def kernel(edge_index, U, V):
    src = edge_index[0].astype(jnp.int32).reshape(_E // 128, 128)
    dst = edge_index[1].astype(jnp.int32).reshape(_E // 128, 128)
    return _sc_call(src, dst, U, V)


# speedup vs baseline: 10.3712x; 3.4345x over previous
import functools

import jax
import jax.numpy as jnp
from jax import lax
from jax.experimental import pallas as pl
from jax.experimental.pallas import tpu as pltpu
from jax.experimental.pallas import tpu_sc as plsc

_E = 327680
_D = 64
_LANES = 16
_NW = 32
_EPW = _E // _NW
_C = 256
_JPC = _C // 128
_NCHUNK = _EPW // _C
_IDX_ROWS = _EPW // 128


def _sc_body(src_hbm, dst_hbm, u_hbm, v_hbm, out_hbm,
             sidx, didx, ubuf0, vbuf0, ubuf1, vbuf1, obuf, sem0, sem1):
    nc = 2
    wid = lax.axis_index("s") * nc + lax.axis_index("c")
    row0 = wid * _IDX_ROWS

    pltpu.sync_copy(src_hbm.at[pl.ds(row0, _IDX_ROWS)], sidx)
    pltpu.sync_copy(dst_hbm.at[pl.ds(row0, _IDX_ROWS)], didx)

    iota = lax.iota(jnp.int32, _LANES)

    def fire(g, ubuf, vbuf, sem):
        j0 = g * _JPC
        for j in range(_JPC):
            pltpu.make_async_copy(
                u_hbm.at[sidx.at[j0 + j]],
                ubuf.at[pl.ds(j * 128, 128)], sem).start()
            pltpu.make_async_copy(
                v_hbm.at[didx.at[j0 + j]],
                vbuf.at[pl.ds(j * 128, 128)], sem).start()

    def wait(g, ubuf, vbuf, sem):
        j0 = g * _JPC
        for j in range(_JPC):
            pltpu.make_async_copy(
                u_hbm.at[sidx.at[j0 + j]],
                ubuf.at[pl.ds(j * 128, 128)], sem).wait()
            pltpu.make_async_copy(
                v_hbm.at[didx.at[j0 + j]],
                vbuf.at[pl.ds(j * 128, 128)], sem).wait()

    def compute(g, ubuf, vbuf):
        out_base = g * _C

        def grp_body(grp, _):
            rows = grp * _LANES + iota

            def d_body(t, acc):
                for k in range(4):
                    d = t * 4 + k
                    cols = (iota + d) & (_D - 1)
                    u = plsc.load_gather(ubuf, [rows, cols])
                    v = plsc.load_gather(vbuf, [rows, cols])
                    x = u * v
                    acc = acc + 1.0 / (1.0 + jnp.exp(-x))
                return acc

            acc = lax.fori_loop(0, _D // 4, d_body,
                                jnp.zeros((_LANES,), jnp.float32))
            obuf[pl.ds(out_base + grp * _LANES, _LANES)] = acc
            return 0

        lax.fori_loop(0, _C // _LANES, grp_body, 0)

    fire(0, ubuf0, vbuf0, sem0)

    def pair_body(gp, _):
        g0 = gp * 2
        fire(g0 + 1, ubuf1, vbuf1, sem1)
        wait(g0, ubuf0, vbuf0, sem0)
        compute(g0, ubuf0, vbuf0)

        @pl.when(g0 + 2 < _NCHUNK)
        def _():
            fire(g0 + 2, ubuf0, vbuf0, sem0)

        wait(g0 + 1, ubuf1, vbuf1, sem1)
        compute(g0 + 1, ubuf1, vbuf1)
        return 0

    lax.fori_loop(0, _NCHUNK // 2, pair_body, 0)

    pltpu.sync_copy(obuf, out_hbm.at[pl.ds(wid * _EPW, _EPW)])


@jax.jit
def _sc_call(src2d, dst2d, u, v):
    f = pl.kernel(
        _sc_body,
        out_type=jax.ShapeDtypeStruct((_E,), jnp.float32),
        mesh=plsc.VectorSubcoreMesh(core_axis_name="c", subcore_axis_name="s"),
        compiler_params=pltpu.CompilerParams(
            needs_layout_passes=False, use_tc_tiling_on_sc=False),
        scratch_types=[
            pltpu.VMEM((_IDX_ROWS, 128), jnp.int32),
            pltpu.VMEM((_IDX_ROWS, 128), jnp.int32),
            pltpu.VMEM((_C, _D), jnp.float32),
            pltpu.VMEM((_C, _D), jnp.float32),
            pltpu.VMEM((_C, _D), jnp.float32),
            pltpu.VMEM((_C, _D), jnp.float32),
            pltpu.VMEM((_EPW,), jnp.float32),
            pltpu.SemaphoreType.DMA,
            pltpu.SemaphoreType.DMA,
        ],
    )
    return f(src2d, dst2d, u, v)


def kernel(edge_index, U, V):
    src = edge_index[0].astype(jnp.int32).reshape(_E // 128, 128)
    dst = edge_index[1].astype(jnp.int32).reshape(_E // 128, 128)
    return _sc_call(src, dst, U, V)
